# baseline (speedup 1.0000x reference)
import functools

import jax
import jax.numpy as jnp
from jax import lax
from jax.experimental import pallas as pl
from jax.experimental.pallas import tpu as pltpu
from jax.experimental.pallas import tpu_sc as plsc

D = 32
K = 20
B = 16384

NC, NS, L = 2, 16, 16
NW = NC * NS
B_PER_W = B // NW
CHUNK = 128
N_CHUNKS = B_PER_W // CHUNK
G_PER_CHUNK = CHUNK // L
IDX_ROWS = CHUNK * K // 128


def _sc_body(cids_hbm, ctx_hbm, in_hbm, out_hbm, s_hbm, p_hbm,
             cid_v, cidx_v, crows, xrows, s_stage, p_stage, sem):
    w = lax.axis_index("s") * NC + lax.axis_index("c")
    wbase = w * B_PER_W

    def chunk_body(c, carry):
        base = wbase + c * CHUNK
        pltpu.sync_copy(cids_hbm.at[pl.ds(base, CHUNK)], cid_v)
        for j in range(IDX_ROWS):
            pltpu.sync_copy(ctx_hbm.at[pl.ds(base * K + j * 128, 128)],
                            cidx_v.at[j])
        cps = [pltpu.async_copy(in_hbm.at[cid_v], crows, sem)]
        for j in range(IDX_ROWS):
            cps.append(pltpu.async_copy(out_hbm.at[cidx_v.at[j]],
                                        xrows.at[pl.ds(j * 128, 128)], sem))
        for cp in cps:
            cp.wait()

        iota = lax.iota(jnp.int32, L)

        def group_body(g, gcarry):
            cbase = iota + g * L
            rbase = iota * K + g * (L * K)
            accs = [jnp.zeros((L,), jnp.float32) for _ in range(K)]
            for d in range(D):
                dcol = jnp.full((L,), d, jnp.int32)
                cd = plsc.load_gather(crows, [cbase, dcol])
                for k in range(K):
                    x = plsc.load_gather(xrows, [rbase + k, dcol])
                    accs[k] = accs[k] + cd * x
            m = accs[0]
            for k in range(1, K):
                m = jnp.maximum(m, accs[k])
            t = accs[0]
            for k in range(1, K):
                t = t + accs[k]
            s = jnp.exp(accs[0] - m)
            for k in range(1, K):
                s = s + jnp.exp(accs[k] - m)
            p = K * m - t
            s_stage[pl.ds(g * L, L)] = s
            p_stage[pl.ds(g * L, L)] = p
            return gcarry

        lax.fori_loop(0, G_PER_CHUNK, group_body, 0)
        pltpu.sync_copy(s_stage, s_hbm.at[pl.ds(base, CHUNK)])
        pltpu.sync_copy(p_stage, p_hbm.at[pl.ds(base, CHUNK)])
        return carry

    lax.fori_loop(0, N_CHUNKS, chunk_body, 0)


_sc_kernel = functools.partial(
    pl.kernel,
    out_type=(jax.ShapeDtypeStruct((B,), jnp.float32),
              jax.ShapeDtypeStruct((B,), jnp.float32)),
    mesh=plsc.VectorSubcoreMesh(core_axis_name="c", subcore_axis_name="s"),
    scratch_types=[
        pltpu.VMEM((CHUNK,), jnp.int32),
        pltpu.VMEM((IDX_ROWS, 128), jnp.int32),
        pltpu.VMEM((CHUNK, D), jnp.float32),
        pltpu.VMEM((CHUNK * K, D), jnp.float32),
        pltpu.VMEM((CHUNK,), jnp.float32),
        pltpu.VMEM((CHUNK,), jnp.float32),
        pltpu.SemaphoreType.DMA,
    ],
    compiler_params=pltpu.CompilerParams(needs_layout_passes=False,
                                         use_tc_tiling_on_sc=False),
)(_sc_body)


def _tc_body(s_ref, p_ref, o_ref):
    lse = jnp.log(s_ref[...])
    loss = (jnp.sum(p_ref[...]) + K * jnp.sum(lse)) / (B * K)
    o_ref[...] = loss[None, None]


def kernel(center_ids, context_ids, in_embed, out_embed):
    ctx_rs = context_ids.reshape(B * K)
    s, p = _sc_kernel(center_ids, ctx_rs, in_embed, out_embed)
    loss2d = pl.pallas_call(
        _tc_body,
        out_shape=jax.ShapeDtypeStruct((1, 1), jnp.float32),
    )(s.reshape(128, 128), p.reshape(128, 128))
    return loss2d[0, 0]
